# baseline (speedup 1.0000x reference)
import functools

import jax
import jax.numpy as jnp
from jax import lax
from jax.experimental import pallas as pl
from jax.experimental.pallas import tpu as pltpu
from jax.experimental.pallas import tpu_sc as plsc

_NC = 2
_NS = 16
_EB = 64
_BN = 1000


def _mm(x, w, b, add=None, stats=False):
    n, k = x.shape
    m = w.shape[1]
    grid = (n // _BN,)
    b2 = b.reshape(1, m)

    def body(*refs):
        if add is not None:
            x_ref, w_ref, b_ref, a_ref = refs[:4]
            orefs = refs[4:]
        else:
            x_ref, w_ref, b_ref = refs[:3]
            orefs = refs[3:]
        y = jnp.dot(x_ref[...], w_ref[...], preferred_element_type=jnp.float32)
        y = y + b_ref[...]
        if add is not None:
            y = y + a_ref[...]
        orefs[0][...] = y
        if stats:
            st_ref = orefs[1]

            @pl.when(pl.program_id(0) == 0)
            def _():
                st_ref[...] = jnp.zeros_like(st_ref)

            s0 = jnp.sum(y, axis=0, keepdims=True)
            s1 = jnp.sum(y * y, axis=0, keepdims=True)
            st_ref[...] += jnp.concatenate(
                [s0, s1, jnp.zeros((6, m), jnp.float32)], axis=0)

    in_specs = [
        pl.BlockSpec((_BN, k), lambda i: (i, 0)),
        pl.BlockSpec((k, m), lambda i: (0, 0)),
        pl.BlockSpec((1, m), lambda i: (0, 0)),
    ]
    args = [x, w, b2]
    if add is not None:
        in_specs.append(pl.BlockSpec((_BN, m), lambda i: (i, 0)))
        args.append(add)
    out_shape = [jax.ShapeDtypeStruct((n, m), jnp.float32)]
    out_specs = [pl.BlockSpec((_BN, m), lambda i: (i, 0))]
    if stats:
        out_shape.append(jax.ShapeDtypeStruct((8, m), jnp.float32))
        out_specs.append(pl.BlockSpec((8, m), lambda i: (0, 0)))
    res = pl.pallas_call(
        body, grid=grid, in_specs=in_specs, out_specs=out_specs,
        out_shape=out_shape)(*args)
    return res if stats else res[0]


def _mm2(hs, ws):
    _, n, _ = hs.shape
    m = ws.shape[2]

    def body(h_ref, w_ref, o_ref):
        o_ref[...] = (
            jnp.dot(h_ref[0], w_ref[0], preferred_element_type=jnp.float32)
            + jnp.dot(h_ref[1], w_ref[1], preferred_element_type=jnp.float32))

    return pl.pallas_call(
        body, grid=(n // _BN,),
        in_specs=[
            pl.BlockSpec((2, _BN, 128), lambda i: (0, i, 0)),
            pl.BlockSpec((2, 128, m), lambda i: (0, 0, 0)),
        ],
        out_specs=pl.BlockSpec((_BN, m), lambda i: (i, 0)),
        out_shape=jax.ShapeDtypeStruct((n, m), jnp.float32))(hs, ws)


def _gnn_mm(hs, ws, p, wn, degp, b, stats=False, shared_wn=False):
    ch, n, _ = hs.shape
    m = ws.shape[2]
    grid = (n // _BN,)
    b2 = b.reshape(1, m)

    def body(*refs):
        if wn is None:
            h_ref, ws_ref, p_ref, d_ref, b_ref = refs[:5]
            orefs = refs[5:]
        else:
            h_ref, ws_ref, p_ref, wn_ref, d_ref, b_ref = refs[:6]
            orefs = refs[6:]
        r = 1.0 / jnp.maximum(d_ref[0, :, 0:1] + d_ref[1, :, 0:1], 1.0)
        z = jnp.zeros((_BN, m), jnp.float32)
        for c in range(ch):
            z += jnp.dot(h_ref[c], ws_ref[c], preferred_element_type=jnp.float32)
        if wn is None:
            z += (p_ref[0] + p_ref[1]) * r
        elif shared_wn:
            q = (p_ref[0] + p_ref[1]) * r
            z += jnp.dot(q, wn_ref[0], preferred_element_type=jnp.float32)
        else:
            for c in range(2):
                z += jnp.dot(p_ref[c] * r, wn_ref[c],
                             preferred_element_type=jnp.float32)
        z = z + b_ref[...]
        orefs[0][...] = z
        if stats:
            st_ref = orefs[1]

            @pl.when(pl.program_id(0) == 0)
            def _():
                st_ref[...] = jnp.zeros_like(st_ref)

            s0 = jnp.sum(z, axis=0, keepdims=True)
            s1 = jnp.sum(z * z, axis=0, keepdims=True)
            st_ref[...] += jnp.concatenate(
                [s0, s1, jnp.zeros((6, m), jnp.float32)], axis=0)

    in_specs = [
        pl.BlockSpec((ch, _BN, 128), lambda i: (0, i, 0)),
        pl.BlockSpec((ch, 128, m), lambda i: (0, 0, 0)),
        pl.BlockSpec((2, _BN, 128), lambda i: (0, i, 0)),
    ]
    args = [hs, ws, p]
    if wn is not None:
        cw = wn.shape[0]
        in_specs.append(pl.BlockSpec((cw, 128, m), lambda i: (0, 0, 0)))
        args.append(wn)
    in_specs += [
        pl.BlockSpec((2, _BN, 128), lambda i: (0, i, 0)),
        pl.BlockSpec((1, m), lambda i: (0, 0)),
    ]
    args += [degp, b2]
    out_shape = [jax.ShapeDtypeStruct((n, m), jnp.float32)]
    out_specs = [pl.BlockSpec((_BN, m), lambda i: (i, 0))]
    if stats:
        out_shape.append(jax.ShapeDtypeStruct((8, m), jnp.float32))
        out_specs.append(pl.BlockSpec((8, m), lambda i: (0, 0)))
    res = pl.pallas_call(
        body, grid=grid, in_specs=in_specs, out_specs=out_specs,
        out_shape=out_shape)(*args)
    return res if stats else res[0]


def _bnrelu(z, a, c, cout):
    n, m = z.shape
    a2 = a.reshape(1, m)
    c2 = c.reshape(1, m)

    def body(z_ref, a_ref, c_ref, o_ref):
        o_ref[0] = jnp.maximum(z_ref[...] * a_ref[...] + c_ref[...], 0.0)

    return pl.pallas_call(
        body, grid=(cout, n // _BN),
        in_specs=[
            pl.BlockSpec((_BN, 128), lambda co, i: (i, co)),
            pl.BlockSpec((1, 128), lambda co, i: (0, co)),
            pl.BlockSpec((1, 128), lambda co, i: (0, co)),
        ],
        out_specs=pl.BlockSpec((1, _BN, 128), lambda co, i: (co, i, 0)),
        out_shape=jax.ShapeDtypeStruct((cout, n, 128), jnp.float32),
    )(z, a2, c2)


def _bn_coef(st, n, g, bt):
    mu = st[0] / n
    var = st[1] / n - mu * mu
    a = g * lax.rsqrt(var + 1e-5)
    return a, bt - mu * a


def _make_seg(n, e, c_split):
    nblk = e // _EB
    stride = _NC * _NS if c_split == 1 else _NS
    assert nblk % (8 * stride) == 0
    nb = nblk // stride
    nb8 = nb // 8
    nacc = n + 8
    rpt = (n // _NS) & ~7
    wtail = n - rpt * _NS
    itail = nacc - rpt * _NS
    mesh = plsc.VectorSubcoreMesh(
        core_axis_name="c", subcore_axis_name="s",
        num_cores=_NC, num_subcores=_NS)
    outs = [jax.ShapeDtypeStruct((2, n, 128), jnp.float32)]
    scratch = (
        [pltpu.VMEM((8, 2, _EB), jnp.int32),
         pltpu.VMEM((4, _EB, 128), jnp.float32),
         pltpu.VMEM_SHARED((nacc, 128), jnp.float32)]
        + [pltpu.SemaphoreType.DMA] * 16
    )

    def body(table, idxpk, zrows, out, idx_v, rows_v, acc, *sems):
        sa, sb, scm = sems[0:8], sems[8:12], sems[12:16]
        sc = lax.axis_index("c")
        t = lax.axis_index("s")
        w = sc * _NS + t if c_split == 1 else t

        ro = t * rpt
        pltpu.sync_copy(zrows, acc.at[pl.ds(ro, rpt)])

        @pl.when(t == 0)
        def _():
            pltpu.sync_copy(zrows.at[pl.ds(0, itail)],
                            acc.at[pl.ds(rpt * _NS, itail)])
        plsc.subcore_barrier()

        def a_start(i, slot):
            pltpu.async_copy(idxpk.at[sc, w + i * stride], idx_v.at[slot],
                             sa[slot])

        def a_wait(i, slot):
            pltpu.make_async_copy(idxpk.at[sc, w + i * stride],
                                  idx_v.at[slot], sa[slot]).wait()

        def b_start(slot, rb):
            pltpu.async_copy(table.at[idx_v.at[slot, 0]], rows_v.at[rb],
                             sb[rb])

        def b_wait(slot, rb):
            pltpu.make_async_copy(table.at[idx_v.at[slot, 0]],
                                  rows_v.at[rb], sb[rb]).wait()

        def c_start(slot, rb):
            pltpu.async_copy(rows_v.at[rb], acc.at[idx_v.at[slot, 1]],
                             scm[rb], add=True)

        def c_wait(slot, rb):
            pltpu.make_async_copy(rows_v.at[rb], acc.at[idx_v.at[slot, 1]],
                                  scm[rb]).wait()

        for k in range(4):
            a_start(k, k)
        a_wait(0, 0)
        b_start(0, 0)
        a_wait(1, 1)
        b_start(1, 1)

        def outer(jj, carry):
            i0 = jj * 8
            for b in range(8):
                i = i0 + b
                rb = b % 4
                b_wait(b, rb)
                c_start(b, rb)
                ns, nrb = (b + 2) % 8, (b + 2) % 4
                cs = (b + 6) % 8

                def issue_next():
                    if b >= 2:
                        c_wait(cs, nrb)
                    else:
                        @pl.when(jj > 0)
                        def _():
                            c_wait(cs, nrb)
                    a_wait(i + 2, ns)
                    b_start(ns, nrb)

                def issue_a():
                    a_start(i + 4, (b + 4) % 8)

                if b < 6:
                    issue_next()
                    if b < 4:
                        issue_a()
                    else:
                        @pl.when(jj < nb8 - 1)
                        def _():
                            issue_a()
                else:
                    @pl.when(jj < nb8 - 1)
                    def _():
                        issue_next()
                        issue_a()
            return carry

        lax.fori_loop(0, nb8, outer, 0)
        for k in range(4):
            c_wait(4 + k, k)
        plsc.subcore_barrier()
        pltpu.sync_copy(acc.at[pl.ds(ro, rpt)], out.at[sc, pl.ds(ro, rpt)])

        @pl.when(t == 0)
        def _():
            pltpu.sync_copy(acc.at[pl.ds(rpt * _NS, wtail)],
                            out.at[sc, pl.ds(rpt * _NS, wtail)])

    return pl.kernel(body, out_type=outs, mesh=mesh, scratch_types=scratch)


def _make_deg(n, e):
    nblk = e // _EB
    stride = _NC * _NS
    assert nblk % (4 * stride) == 0
    nb = nblk // stride
    nb4 = nb // 4
    nacc = n + 8
    rpt = (n // _NS) & ~7
    wtail = n - rpt * _NS
    itail = nacc - rpt * _NS
    mesh = plsc.VectorSubcoreMesh(
        core_axis_name="c", subcore_axis_name="s",
        num_cores=_NC, num_subcores=_NS)
    outs = [jax.ShapeDtypeStruct((2, n, 128), jnp.float32)]
    scratch = (
        [pltpu.VMEM((4, _EB), jnp.int32),
         pltpu.VMEM((_EB, 128), jnp.float32),
         pltpu.VMEM_SHARED((nacc, 128), jnp.float32)]
        + [pltpu.SemaphoreType.DMA] * 8
    )

    def body(idxpk, zrows, onesr, out, didx_v, ones_v, acc, *sems):
        sa, scm = sems[0:4], sems[4:8]
        sc = lax.axis_index("c")
        t = lax.axis_index("s")
        w = sc * _NS + t

        ro = t * rpt
        pltpu.sync_copy(zrows, acc.at[pl.ds(ro, rpt)])
        pltpu.sync_copy(onesr, ones_v)

        @pl.when(t == 0)
        def _():
            pltpu.sync_copy(zrows.at[pl.ds(0, itail)],
                            acc.at[pl.ds(rpt * _NS, itail)])
        plsc.subcore_barrier()

        def a_start(i, slot):
            pltpu.async_copy(idxpk.at[sc, w + i * stride, 1],
                             didx_v.at[slot], sa[slot])

        def a_wait(i, slot):
            pltpu.make_async_copy(idxpk.at[sc, w + i * stride, 1],
                                  didx_v.at[slot], sa[slot]).wait()

        def c_start(slot):
            pltpu.async_copy(ones_v, acc.at[didx_v.at[slot]], scm[slot],
                             add=True)

        def c_wait(slot):
            pltpu.make_async_copy(ones_v, acc.at[didx_v.at[slot]],
                                  scm[slot]).wait()

        a_start(0, 0)
        a_start(1, 1)

        def outer(jj, carry):
            i0 = jj * 4
            for b in range(4):
                i = i0 + b
                a_wait(i, b)
                cs = (b + 2) % 4
                if b >= 2:
                    c_wait(cs)
                else:
                    @pl.when(jj > 0)
                    def _():
                        c_wait(cs)
                c_start(b)
                if b < 2:
                    a_start(i + 2, cs)
                else:
                    @pl.when(jj < nb4 - 1)
                    def _():
                        a_start(i + 2, cs)
            return carry

        lax.fori_loop(0, nb4, outer, 0)
        c_wait(2)
        c_wait(3)
        plsc.subcore_barrier()
        pltpu.sync_copy(acc.at[pl.ds(ro, rpt)], out.at[sc, pl.ds(ro, rpt)])

        @pl.when(t == 0)
        def _():
            pltpu.sync_copy(acc.at[pl.ds(rpt * _NS, wtail)],
                            out.at[sc, pl.ds(rpt * _NS, wtail)])

    return pl.kernel(body, out_type=outs, mesh=mesh, scratch_types=scratch)


def _seg_sum(table, idxpk, c_split):
    n = table.shape[0] // c_split
    e = idxpk.shape[1] * _EB
    rpt = (n // _NS) & ~7
    zrows = jnp.zeros((rpt, 128), jnp.float32)
    return _make_seg(n, e, c_split)(table, idxpk, zrows)[0]


def _deg(idxpk, n):
    e = idxpk.shape[1] * _EB
    rpt = (n // _NS) & ~7
    zrows = jnp.zeros((rpt, 128), jnp.float32)
    onesr = jnp.ones((_EB, 128), jnp.float32)
    return _make_deg(n, e)(idxpk, zrows, onesr)[0]


def kernel(x0, x1, edge_index, enc_params, gnn_params):
    n = x0.shape[0]
    src = edge_index[0].astype(jnp.int32)
    dst = edge_index[1].astype(jnp.int32)
    grain = 8 * _NC * _NS * _EB
    ep = -(-dst.shape[0] // grain) * grain
    pe = ep - dst.shape[0]
    srcp = jnp.concatenate([src, jnp.zeros((pe,), jnp.int32)])
    dstp = jnp.concatenate([dst, jnp.full((pe,), n, jnp.int32)])
    srcr = srcp.reshape(-1, _EB)
    dstr = dstp.reshape(-1, _EB)
    pk0 = jnp.stack([srcr, dstr], axis=1)
    pk_same = jnp.stack([pk0, pk0])

    h = None
    for x, prm in ((x0, enc_params[0]), (x1, enc_params[1])):
        w1, b1, g1, bt1, w2, b2, g2, bt2, wd, bd = prm
        y1, st1 = _mm(x, w1, b1, stats=True)
        a1, c1 = _bn_coef(st1, n, g1, bt1)
        y2, st2 = _mm(y1, a1[:, None] * w2, c1 @ w2 + b2, stats=True)
        a2, c2 = _bn_coef(st2, n, g2, bt2)
        h = _mm(y2, a2[:, None] * wd, c2 @ wd + bd, add=h)

    ws0, wn0, b0, g0, bt0 = gnn_params[0]
    degp = _deg(pk_same, n)
    p0 = _seg_sum(h, pk_same, c_split=1)
    z0, s0 = _gnn_mm(h[None], ws0[None], p0, wn0[None], degp, b0,
                     stats=True, shared_wn=True)
    a0, c0 = _bn_coef(s0, n, g0, bt0)
    h1 = _bnrelu(z0, a0, c0, cout=2)

    ws1, wn1, b1, g1, bt1 = gnn_params[1]
    t1 = _mm2(h1, wn1.reshape(2, 128, 128))
    p1 = _seg_sum(t1, pk_same, c_split=1)
    z1, s1 = _gnn_mm(h1, ws1.reshape(2, 128, 128), p1,
                     None, degp, b1, stats=True)
    a1, c1 = _bn_coef(s1, n, g1, bt1)
    h2 = _bnrelu(z1, a1, c1, cout=1).reshape(n, 128)

    ws2, wn2, b2 = gnn_params[2]
    p2 = _seg_sum(h2, pk_same, c_split=1)
    out = _gnn_mm(h2[None], ws2[None], p2, wn2[None], degp, b2,
                  shared_wn=True)
    return out


# speedup vs baseline: 1.0716x; 1.0716x over previous
import functools

import jax
import jax.numpy as jnp
from jax import lax
from jax.experimental import pallas as pl
from jax.experimental.pallas import tpu as pltpu
from jax.experimental.pallas import tpu_sc as plsc

_NC = 2
_NS = 16
_EB = 64
_BN = 1000


def _mm(x, w, b, add=None, stats=False):
    n, k = x.shape
    m = w.shape[1]
    grid = (n // _BN,)
    b2 = b.reshape(1, m)

    def body(*refs):
        if add is not None:
            x_ref, w_ref, b_ref, a_ref = refs[:4]
            orefs = refs[4:]
        else:
            x_ref, w_ref, b_ref = refs[:3]
            orefs = refs[3:]
        y = jnp.dot(x_ref[...], w_ref[...], preferred_element_type=jnp.float32)
        y = y + b_ref[...]
        if add is not None:
            y = y + a_ref[...]
        orefs[0][...] = y
        if stats:
            st_ref = orefs[1]

            @pl.when(pl.program_id(0) == 0)
            def _():
                st_ref[...] = jnp.zeros_like(st_ref)

            s0 = jnp.sum(y, axis=0, keepdims=True)
            s1 = jnp.sum(y * y, axis=0, keepdims=True)
            st_ref[...] += jnp.concatenate(
                [s0, s1, jnp.zeros((6, m), jnp.float32)], axis=0)

    in_specs = [
        pl.BlockSpec((_BN, k), lambda i: (i, 0)),
        pl.BlockSpec((k, m), lambda i: (0, 0)),
        pl.BlockSpec((1, m), lambda i: (0, 0)),
    ]
    args = [x, w, b2]
    if add is not None:
        in_specs.append(pl.BlockSpec((_BN, m), lambda i: (i, 0)))
        args.append(add)
    out_shape = [jax.ShapeDtypeStruct((n, m), jnp.float32)]
    out_specs = [pl.BlockSpec((_BN, m), lambda i: (i, 0))]
    if stats:
        out_shape.append(jax.ShapeDtypeStruct((8, m), jnp.float32))
        out_specs.append(pl.BlockSpec((8, m), lambda i: (0, 0)))
    res = pl.pallas_call(
        body, grid=grid, in_specs=in_specs, out_specs=out_specs,
        out_shape=out_shape)(*args)
    return res if stats else res[0]


def _mm2(hs, ws):
    _, n, _ = hs.shape
    m = ws.shape[2]

    def body(h_ref, w_ref, o_ref):
        o_ref[...] = (
            jnp.dot(h_ref[0], w_ref[0], preferred_element_type=jnp.float32)
            + jnp.dot(h_ref[1], w_ref[1], preferred_element_type=jnp.float32))

    return pl.pallas_call(
        body, grid=(n // _BN,),
        in_specs=[
            pl.BlockSpec((2, _BN, 128), lambda i: (0, i, 0)),
            pl.BlockSpec((2, 128, m), lambda i: (0, 0, 0)),
        ],
        out_specs=pl.BlockSpec((_BN, m), lambda i: (i, 0)),
        out_shape=jax.ShapeDtypeStruct((n, m), jnp.float32))(hs, ws)


def _gnn_mm(hs, ws, p, wn, degp, b, stats=False, shared_wn=False):
    ch, n, _ = hs.shape
    m = ws.shape[2]
    grid = (n // _BN,)
    b2 = b.reshape(1, m)

    def body(*refs):
        if wn is None:
            h_ref, ws_ref, p_ref, d_ref, b_ref = refs[:5]
            orefs = refs[5:]
        else:
            h_ref, ws_ref, p_ref, wn_ref, d_ref, b_ref = refs[:6]
            orefs = refs[6:]
        r = 1.0 / jnp.maximum(d_ref[0, :, 0:1] + d_ref[1, :, 0:1], 1.0)
        z = jnp.zeros((_BN, m), jnp.float32)
        for c in range(ch):
            z += jnp.dot(h_ref[c], ws_ref[c], preferred_element_type=jnp.float32)
        if wn is None:
            z += (p_ref[0] + p_ref[1]) * r
        elif shared_wn:
            q = (p_ref[0] + p_ref[1]) * r
            z += jnp.dot(q, wn_ref[0], preferred_element_type=jnp.float32)
        else:
            for c in range(2):
                z += jnp.dot(p_ref[c] * r, wn_ref[c],
                             preferred_element_type=jnp.float32)
        z = z + b_ref[...]
        orefs[0][...] = z
        if stats:
            st_ref = orefs[1]

            @pl.when(pl.program_id(0) == 0)
            def _():
                st_ref[...] = jnp.zeros_like(st_ref)

            s0 = jnp.sum(z, axis=0, keepdims=True)
            s1 = jnp.sum(z * z, axis=0, keepdims=True)
            st_ref[...] += jnp.concatenate(
                [s0, s1, jnp.zeros((6, m), jnp.float32)], axis=0)

    in_specs = [
        pl.BlockSpec((ch, _BN, 128), lambda i: (0, i, 0)),
        pl.BlockSpec((ch, 128, m), lambda i: (0, 0, 0)),
        pl.BlockSpec((2, _BN, 128), lambda i: (0, i, 0)),
    ]
    args = [hs, ws, p]
    if wn is not None:
        cw = wn.shape[0]
        in_specs.append(pl.BlockSpec((cw, 128, m), lambda i: (0, 0, 0)))
        args.append(wn)
    in_specs += [
        pl.BlockSpec((2, _BN, 128), lambda i: (0, i, 0)),
        pl.BlockSpec((1, m), lambda i: (0, 0)),
    ]
    args += [degp, b2]
    out_shape = [jax.ShapeDtypeStruct((n, m), jnp.float32)]
    out_specs = [pl.BlockSpec((_BN, m), lambda i: (i, 0))]
    if stats:
        out_shape.append(jax.ShapeDtypeStruct((8, m), jnp.float32))
        out_specs.append(pl.BlockSpec((8, m), lambda i: (0, 0)))
    res = pl.pallas_call(
        body, grid=grid, in_specs=in_specs, out_specs=out_specs,
        out_shape=out_shape)(*args)
    return res if stats else res[0]


def _bnrelu(z, a, c, cout):
    n, m = z.shape
    a2 = a.reshape(1, m)
    c2 = c.reshape(1, m)

    def body(z_ref, a_ref, c_ref, o_ref):
        o_ref[0] = jnp.maximum(z_ref[...] * a_ref[...] + c_ref[...], 0.0)

    return pl.pallas_call(
        body, grid=(cout, n // _BN),
        in_specs=[
            pl.BlockSpec((_BN, 128), lambda co, i: (i, co)),
            pl.BlockSpec((1, 128), lambda co, i: (0, co)),
            pl.BlockSpec((1, 128), lambda co, i: (0, co)),
        ],
        out_specs=pl.BlockSpec((1, _BN, 128), lambda co, i: (co, i, 0)),
        out_shape=jax.ShapeDtypeStruct((cout, n, 128), jnp.float32),
    )(z, a2, c2)


def _bn_coef(st, n, g, bt):
    mu = st[0] / n
    var = st[1] / n - mu * mu
    a = g * lax.rsqrt(var + 1e-5)
    return a, bt - mu * a


def _make_seg(n, e, c_split):
    nblk = e // _EB
    stride = _NC * _NS if c_split == 1 else _NS
    assert nblk % (8 * stride) == 0
    nb = nblk // stride
    nb8 = nb // 8
    nacc = n + 8
    rpt = (n // _NS) & ~7
    wtail = n - rpt * _NS
    itail = nacc - rpt * _NS
    mesh = plsc.VectorSubcoreMesh(
        core_axis_name="c", subcore_axis_name="s",
        num_cores=_NC, num_subcores=_NS)
    outs = [jax.ShapeDtypeStruct((2, n, 128), jnp.float32)]
    scratch = (
        [pltpu.VMEM((8, 2, _EB), jnp.int32),
         pltpu.VMEM((4, _EB, 128), jnp.float32),
         pltpu.VMEM_SHARED((nacc, 128), jnp.float32)]
        + [pltpu.SemaphoreType.DMA] * 16
    )

    def body(table, idxpk, zrows, out, idx_v, rows_v, acc, *sems):
        sa, sb, scm = sems[0:8], sems[8:12], sems[12:16]
        sc = lax.axis_index("c")
        t = lax.axis_index("s")
        w = sc * _NS + t if c_split == 1 else t

        ro = t * rpt
        pltpu.sync_copy(zrows, acc.at[pl.ds(ro, rpt)])

        @pl.when(t == 0)
        def _():
            pltpu.sync_copy(zrows.at[pl.ds(0, itail)],
                            acc.at[pl.ds(rpt * _NS, itail)])
        plsc.subcore_barrier()

        def a_start(i, slot):
            pltpu.async_copy(idxpk.at[sc, w + i * stride], idx_v.at[slot],
                             sa[slot])

        def a_wait(i, slot):
            pltpu.make_async_copy(idxpk.at[sc, w + i * stride],
                                  idx_v.at[slot], sa[slot]).wait()

        def b_start(slot, rb):
            pltpu.async_copy(table.at[idx_v.at[slot, 0]], rows_v.at[rb],
                             sb[rb])

        def b_wait(slot, rb):
            pltpu.make_async_copy(table.at[idx_v.at[slot, 0]],
                                  rows_v.at[rb], sb[rb]).wait()

        def c_start(slot, rb):
            pltpu.async_copy(rows_v.at[rb], acc.at[idx_v.at[slot, 1]],
                             scm[rb], add=True)

        def c_wait(slot, rb):
            pltpu.make_async_copy(rows_v.at[rb], acc.at[idx_v.at[slot, 1]],
                                  scm[rb]).wait()

        for k in range(4):
            a_start(k, k)
        a_wait(0, 0)
        b_start(0, 0)
        a_wait(1, 1)
        b_start(1, 1)

        def outer(jj, carry):
            i0 = jj * 8
            for b in range(8):
                i = i0 + b
                rb = b % 4
                b_wait(b, rb)
                c_start(b, rb)
                ns, nrb = (b + 2) % 8, (b + 2) % 4
                cs = (b + 6) % 8

                def issue_next():
                    if b >= 2:
                        c_wait(cs, nrb)
                    else:
                        @pl.when(jj > 0)
                        def _():
                            c_wait(cs, nrb)
                    a_wait(i + 2, ns)
                    b_start(ns, nrb)

                def issue_a():
                    a_start(i + 4, (b + 4) % 8)

                if b < 6:
                    issue_next()
                    if b < 4:
                        issue_a()
                    else:
                        @pl.when(jj < nb8 - 1)
                        def _():
                            issue_a()
                else:
                    @pl.when(jj < nb8 - 1)
                    def _():
                        issue_next()
                        issue_a()
            return carry

        lax.fori_loop(0, nb8, outer, 0)
        for k in range(4):
            c_wait(4 + k, k)
        plsc.subcore_barrier()
        pltpu.sync_copy(acc.at[pl.ds(ro, rpt)], out.at[sc, pl.ds(ro, rpt)])

        @pl.when(t == 0)
        def _():
            pltpu.sync_copy(acc.at[pl.ds(rpt * _NS, wtail)],
                            out.at[sc, pl.ds(rpt * _NS, wtail)])

    return pl.kernel(body, out_type=outs, mesh=mesh, scratch_types=scratch)


def _make_deg(n, e):
    nblk = e // _EB
    stride = _NC * _NS
    assert nblk % (4 * stride) == 0
    nb = nblk // stride
    nb4 = nb // 4
    nacc = n + 8
    rpt = (n // _NS) & ~7
    wtail = n - rpt * _NS
    itail = nacc - rpt * _NS
    mesh = plsc.VectorSubcoreMesh(
        core_axis_name="c", subcore_axis_name="s",
        num_cores=_NC, num_subcores=_NS)
    outs = [jax.ShapeDtypeStruct((2, n, 128), jnp.float32)]
    scratch = (
        [pltpu.VMEM((4, _EB), jnp.int32),
         pltpu.VMEM((_EB, 128), jnp.float32),
         pltpu.VMEM_SHARED((nacc, 128), jnp.float32)]
        + [pltpu.SemaphoreType.DMA] * 8
    )

    def body(idxpk, zrows, onesr, out, didx_v, ones_v, acc, *sems):
        sa, scm = sems[0:4], sems[4:8]
        sc = lax.axis_index("c")
        t = lax.axis_index("s")
        w = sc * _NS + t

        ro = t * rpt
        pltpu.sync_copy(zrows, acc.at[pl.ds(ro, rpt)])
        pltpu.sync_copy(onesr, ones_v)

        @pl.when(t == 0)
        def _():
            pltpu.sync_copy(zrows.at[pl.ds(0, itail)],
                            acc.at[pl.ds(rpt * _NS, itail)])
        plsc.subcore_barrier()

        def a_start(i, slot):
            pltpu.async_copy(idxpk.at[sc, w + i * stride, 1],
                             didx_v.at[slot], sa[slot])

        def a_wait(i, slot):
            pltpu.make_async_copy(idxpk.at[sc, w + i * stride, 1],
                                  didx_v.at[slot], sa[slot]).wait()

        def c_start(slot):
            pltpu.async_copy(ones_v, acc.at[didx_v.at[slot]], scm[slot],
                             add=True)

        def c_wait(slot):
            pltpu.make_async_copy(ones_v, acc.at[didx_v.at[slot]],
                                  scm[slot]).wait()

        a_start(0, 0)
        a_start(1, 1)

        def outer(jj, carry):
            i0 = jj * 4
            for b in range(4):
                i = i0 + b
                a_wait(i, b)
                cs = (b + 2) % 4
                if b >= 2:
                    c_wait(cs)
                else:
                    @pl.when(jj > 0)
                    def _():
                        c_wait(cs)
                c_start(b)
                if b < 2:
                    a_start(i + 2, cs)
                else:
                    @pl.when(jj < nb4 - 1)
                    def _():
                        a_start(i + 2, cs)
            return carry

        lax.fori_loop(0, nb4, outer, 0)
        c_wait(2)
        c_wait(3)
        plsc.subcore_barrier()
        pltpu.sync_copy(acc.at[pl.ds(ro, rpt)], out.at[sc, pl.ds(ro, rpt)])

        @pl.when(t == 0)
        def _():
            pltpu.sync_copy(acc.at[pl.ds(rpt * _NS, wtail)],
                            out.at[sc, pl.ds(rpt * _NS, wtail)])

    return pl.kernel(body, out_type=outs, mesh=mesh, scratch_types=scratch)


def _seg_sum(table, idxpk, c_split):
    n = table.shape[0] // 2
    e = idxpk.shape[1] * _EB
    rpt = (n // _NS) & ~7
    zrows = jnp.zeros((rpt, 128), jnp.float32)
    return _make_seg(n, e, c_split)(table, idxpk, zrows)[0]


def _deg(idxpk, n):
    e = idxpk.shape[1] * _EB
    rpt = (n // _NS) & ~7
    zrows = jnp.zeros((rpt, 128), jnp.float32)
    onesr = jnp.ones((_EB, 128), jnp.float32)
    return _make_deg(n, e)(idxpk, zrows, onesr)[0]


def kernel(x0, x1, edge_index, enc_params, gnn_params):
    n = x0.shape[0]
    src = edge_index[0].astype(jnp.int32)
    dst = edge_index[1].astype(jnp.int32)
    grain = 8 * _NC * _NS * _EB
    ep = -(-dst.shape[0] // grain) * grain
    pe = ep - dst.shape[0]
    srcp = jnp.concatenate([src, jnp.zeros((pe,), jnp.int32)])
    dstp = jnp.concatenate([dst, jnp.full((pe,), n, jnp.int32)])
    srcr = srcp.reshape(-1, _EB)
    dstr = dstp.reshape(-1, _EB)
    pk0 = jnp.stack([srcr, dstr], axis=1)
    pk_same = jnp.stack([pk0, pk0])
    pk_off = jnp.stack([pk0, jnp.stack([srcr + n, dstr], axis=1)])

    h = None
    for x, prm in ((x0, enc_params[0]), (x1, enc_params[1])):
        w1, b1, g1, bt1, w2, b2, g2, bt2, wd, bd = prm
        y1, st1 = _mm(x, w1, b1, stats=True)
        a1, c1 = _bn_coef(st1, n, g1, bt1)
        y2, st2 = _mm(y1, a1[:, None] * w2, c1 @ w2 + b2, stats=True)
        a2, c2 = _bn_coef(st2, n, g2, bt2)
        h = _mm(y2, a2[:, None] * wd, c2 @ wd + bd, add=h)

    ws0, wn0, b0, g0, bt0 = gnn_params[0]
    degp = _deg(pk_same, n)
    p0 = _seg_sum(jnp.concatenate([h, h]), pk_off, c_split=1)
    z0, s0 = _gnn_mm(h[None], ws0[None], p0, wn0[None], degp, b0,
                     stats=True, shared_wn=True)
    a0, c0 = _bn_coef(s0, n, g0, bt0)
    h1 = _bnrelu(z0, a0, c0, cout=2)

    ws1, wn1, b1, g1, bt1 = gnn_params[1]
    t1 = _mm2(h1, wn1.reshape(2, 128, 128))
    p1 = _seg_sum(jnp.concatenate([t1, t1]), pk_off, c_split=1)
    z1, s1 = _gnn_mm(h1, ws1.reshape(2, 128, 128), p1,
                     None, degp, b1, stats=True)
    a1, c1 = _bn_coef(s1, n, g1, bt1)
    h2 = _bnrelu(z1, a1, c1, cout=1).reshape(n, 128)

    ws2, wn2, b2 = gnn_params[2]
    p2 = _seg_sum(jnp.concatenate([h2, h2]), pk_off, c_split=1)
    out = _gnn_mm(h2[None], ws2[None], p2, wn2[None], degp, b2,
                  shared_wn=True)
    return out
